# Initial kernel scaffold; baseline (speedup 1.0000x reference)
import jax
import jax.numpy as jnp
from jax.experimental import pallas as pl


def kernel(snippets, times, labels, offsets):
    raise NotImplementedError("write your pallas kernel here")


# speedup vs baseline: 2.6436x; 2.6436x over previous
import functools

import jax
import jax.numpy as jnp
from jax import lax
from jax.experimental import pallas as pl
from jax.experimental.pallas import tpu as pltpu
from jax.experimental.pallas import tpu_sc as plsc

N = 200000
T = 128
K = 64
L = 16
NW = 32
CHUNK = 400
NCHUNKS = N // CHUNK
GROUPS = CHUNK // L
MAX_CHUNKS_PER_W = -(-NCHUNKS // NW)


def _body(snips, times, labels, offsets, aligned_out, times2_out,
          in_v, out_v, lab_v, tim_v, t2_v, off_v):
    w = lax.axis_index("s") * 2 + lax.axis_index("c")
    pltpu.sync_copy(offsets, off_v)
    lanes = lax.iota(jnp.int32, L)

    def chunk_body(ci, _):
        chunk = w + NW * ci

        @pl.when(chunk < NCHUNKS)
        def _():
            r0 = chunk * CHUNK
            pltpu.sync_copy(snips.at[pl.ds(r0, CHUNK)], in_v)
            pltpu.sync_copy(labels.at[pl.ds(r0, CHUNK)], lab_v)
            pltpu.sync_copy(times.at[pl.ds(r0, CHUNK)], tim_v)

            def group_body(g, _):
                gb = g * L
                lab = lab_v[pl.ds(gb, L)]
                mask = lab > 0
                s = plsc.load_gather(off_v, [jnp.maximum(lab - 1, 0)])
                tim = tim_v[pl.ds(gb, L)]
                t2_v[pl.ds(gb, L)] = jnp.where(
                    mask, tim - s.astype(jnp.float32), 0.0)
                rows = lanes + gb
                col0 = (T - s) & (T - 1)

                def col_body(i, carry):
                    colv, jv = carry
                    c, j = colv, jv
                    for _u in range(8):
                        val = plsc.load_gather(in_v, [rows, c])
                        val = jnp.where(mask, val, 0.0)
                        plsc.store_scatter(out_v, [rows, j], val)
                        c = (c + 1) & (T - 1)
                        j = j + 1
                    return c, j

                lax.fori_loop(0, T // 8, col_body,
                              (col0, jnp.zeros((L,), jnp.int32)))
                return 0

            lax.fori_loop(0, GROUPS, group_body, 0)
            pltpu.sync_copy(out_v, aligned_out.at[pl.ds(r0, CHUNK)])
            pltpu.sync_copy(t2_v, times2_out.at[pl.ds(r0, CHUNK)])
        return 0

    lax.fori_loop(0, MAX_CHUNKS_PER_W, chunk_body, 0)


@jax.jit
def kernel(snippets, times, labels, offsets):
    mesh = plsc.VectorSubcoreMesh(core_axis_name="c", subcore_axis_name="s")
    f = pl.kernel(
        _body,
        out_type=(
            jax.ShapeDtypeStruct((N, T), jnp.float32),
            jax.ShapeDtypeStruct((N,), jnp.float32),
        ),
        mesh=mesh,
        compiler_params=pltpu.CompilerParams(needs_layout_passes=False),
        scratch_types=[
            pltpu.VMEM((CHUNK, T), jnp.float32),
            pltpu.VMEM((CHUNK, T), jnp.float32),
            pltpu.VMEM((CHUNK,), jnp.int32),
            pltpu.VMEM((CHUNK,), jnp.float32),
            pltpu.VMEM((CHUNK,), jnp.float32),
            pltpu.VMEM((K,), jnp.int32),
        ],
    )
    return f(snippets, times, labels, offsets)


# speedup vs baseline: 3.7333x; 1.4122x over previous
import functools

import jax
import jax.numpy as jnp
from jax import lax
from jax.experimental import pallas as pl
from jax.experimental.pallas import tpu as pltpu
from jax.experimental.pallas import tpu_sc as plsc

N = 200000
T = 128
K = 64
L = 16
NW = 32
CHUNK = 400
NCHUNKS = N // CHUNK
GROUPS = CHUNK // L
MAX_CHUNKS_PER_W = -(-NCHUNKS // NW)


def _body(snips, times, labels, offsets, aligned_out, times2_out,
          in_v, out_v, lab_v, tim_v, t2_v, off_v):
    w = lax.axis_index("s") * 2 + lax.axis_index("c")
    pltpu.sync_copy(offsets, off_v)
    lanes = lax.iota(jnp.int32, L)

    def chunk_body(ci, _):
        chunk = w + NW * ci

        @pl.when(chunk < NCHUNKS)
        def _():
            r0 = chunk * CHUNK
            pltpu.sync_copy(snips.at[pl.ds(r0, CHUNK)], in_v)
            pltpu.sync_copy(labels.at[pl.ds(r0, CHUNK)], lab_v)
            pltpu.sync_copy(times.at[pl.ds(r0, CHUNK)], tim_v)

            @plsc.parallel_loop(0, CHUNK, step=L)
            def _group(gb):
                lab = lab_v[pl.ds(gb, L)]
                mask = lab > 0
                s = plsc.load_gather(off_v, [jnp.maximum(lab - 1, 0)])
                tim = tim_v[pl.ds(gb, L)]
                t2_v[pl.ds(gb, L)] = jnp.where(
                    mask, tim - s.astype(jnp.float32), 0.0)
                rows = lanes + gb
                t = (T - s) & (T - 1)

                @plsc.parallel_loop(0, T, unroll=8)
                def _col(j):
                    jv = jnp.full((L,), j, jnp.int32)
                    col = (jv + t) & (T - 1)
                    val = plsc.load_gather(in_v, [rows, col])
                    val = jnp.where(mask, val, 0.0)
                    plsc.store_scatter(out_v, [rows, jv], val)
            pltpu.sync_copy(out_v, aligned_out.at[pl.ds(r0, CHUNK)])
            pltpu.sync_copy(t2_v, times2_out.at[pl.ds(r0, CHUNK)])
        return 0

    lax.fori_loop(0, MAX_CHUNKS_PER_W, chunk_body, 0)


@jax.jit
def kernel(snippets, times, labels, offsets):
    mesh = plsc.VectorSubcoreMesh(core_axis_name="c", subcore_axis_name="s")
    f = pl.kernel(
        _body,
        out_type=(
            jax.ShapeDtypeStruct((N, T), jnp.float32),
            jax.ShapeDtypeStruct((N,), jnp.float32),
        ),
        mesh=mesh,
        compiler_params=pltpu.CompilerParams(needs_layout_passes=False),
        scratch_types=[
            pltpu.VMEM((CHUNK, T), jnp.float32),
            pltpu.VMEM((CHUNK, T), jnp.float32),
            pltpu.VMEM((CHUNK,), jnp.int32),
            pltpu.VMEM((CHUNK,), jnp.float32),
            pltpu.VMEM((CHUNK,), jnp.float32),
            pltpu.VMEM((K,), jnp.int32),
        ],
    )
    return f(snippets, times, labels, offsets)


# speedup vs baseline: 4.5626x; 1.2221x over previous
import jax
import jax.numpy as jnp
from jax import lax
from jax.experimental import pallas as pl
from jax.experimental.pallas import tpu as pltpu
from jax.experimental.pallas import tpu_sc as plsc

N = 200000
T = 128
K = 64
L = 16
NW = 32
CHUNK = 160
NCHUNKS = N // CHUNK
MAXC = -(-NCHUNKS // NW)


def _body(snips, times, labels, offsets, aligned_out, times2_out,
          in_v0, in_v1, out_v0, out_v1, lab_v0, lab_v1,
          tim_v0, tim_v1, t2_v0, t2_v1, off_v,
          si0, si1, sl0, sl1, st0, st1, so0, so1, sq0, sq1):
    w = lax.axis_index("s") * 2 + lax.axis_index("c")
    pltpu.sync_copy(offsets, off_v)
    lanes = lax.iota(jnp.int32, L)
    in_v = (in_v0, in_v1)
    out_v = (out_v0, out_v1)
    lab_v = (lab_v0, lab_v1)
    tim_v = (tim_v0, tim_v1)
    t2_v = (t2_v0, t2_v1)
    in_sems = (si0, si1)
    lab_sems = (sl0, sl1)
    tim_sems = (st0, st1)
    out_sems = (so0, so1)
    t2_sems = (sq0, sq1)

    def start_in(ci, p):
        r0 = (w + NW * ci) * CHUNK
        pltpu.async_copy(snips.at[pl.ds(r0, CHUNK)], in_v[p], in_sems[p])
        pltpu.async_copy(labels.at[pl.ds(r0, CHUNK)], lab_v[p], lab_sems[p])
        pltpu.async_copy(times.at[pl.ds(r0, CHUNK)], tim_v[p], tim_sems[p])

    def wait_in(p):
        pltpu.make_async_copy(snips.at[pl.ds(0, CHUNK)], in_v[p],
                              in_sems[p]).wait()
        pltpu.make_async_copy(labels.at[pl.ds(0, CHUNK)], lab_v[p],
                              lab_sems[p]).wait()
        pltpu.make_async_copy(times.at[pl.ds(0, CHUNK)], tim_v[p],
                              tim_sems[p]).wait()

    def start_out(ci, p):
        r0 = (w + NW * ci) * CHUNK
        pltpu.async_copy(out_v[p], aligned_out.at[pl.ds(r0, CHUNK)],
                         out_sems[p])
        pltpu.async_copy(t2_v[p], times2_out.at[pl.ds(r0, CHUNK)],
                         t2_sems[p])

    def wait_out(p):
        pltpu.make_async_copy(out_v[p], aligned_out.at[pl.ds(0, CHUNK)],
                              out_sems[p]).wait()
        pltpu.make_async_copy(t2_v[p], times2_out.at[pl.ds(0, CHUNK)],
                              t2_sems[p]).wait()

    def compute(p):
        @plsc.parallel_loop(0, CHUNK, step=L)
        def _group(gb):
            lab = lab_v[p][pl.ds(gb, L)]
            mask = lab > 0
            s = plsc.load_gather(off_v, [jnp.maximum(lab - 1, 0)])
            tim = tim_v[p][pl.ds(gb, L)]
            t2_v[p][pl.ds(gb, L)] = jnp.where(
                mask, tim - s.astype(jnp.float32), 0.0)
            rows = lanes + gb
            t = (T - s) & (T - 1)

            @plsc.parallel_loop(0, T, unroll=8)
            def _col(j):
                jv = jnp.full((L,), j, jnp.int32)
                col = (jv + t) & (T - 1)
                val = plsc.load_gather(in_v[p], [rows, col])
                val = jnp.where(mask, val, 0.0)
                plsc.store_scatter(out_v[p], [rows, jv], val)

    def step(ci, p, first, last_valid=None, next_valid=None):
        wait_in(p)
        if next_valid is None:
            start_in(ci + 1, 1 - p)
        elif next_valid is not False:
            @pl.when(next_valid)
            def _():
                start_in(ci + 1, 1 - p)
        if not first:
            wait_out(p)
        compute(p)
        start_out(ci, p)

    start_in(0, 0)
    step(0, 0, first=True)
    step(1, 1, first=True)

    def pair(i, _):
        ci = 2 + 2 * i
        step(ci, 0, first=False)
        step(ci + 1, 1, first=False)
        return 0

    lax.fori_loop(0, (MAXC - 4) // 2, pair, 0)

    last = w + NW * 39 < NCHUNKS
    step(38, 0, first=False, next_valid=last)

    @pl.when(last)
    def _():
        wait_in(1)
        wait_out(1)
        compute(1)
        start_out(39, 1)

    wait_out(0)
    wait_out(1)


@jax.jit
def kernel(snippets, times, labels, offsets):
    mesh = plsc.VectorSubcoreMesh(core_axis_name="c", subcore_axis_name="s")
    f = pl.kernel(
        _body,
        out_type=(
            jax.ShapeDtypeStruct((N, T), jnp.float32),
            jax.ShapeDtypeStruct((N,), jnp.float32),
        ),
        mesh=mesh,
        compiler_params=pltpu.CompilerParams(needs_layout_passes=False),
        scratch_types=[
            pltpu.VMEM((CHUNK, T), jnp.float32),
            pltpu.VMEM((CHUNK, T), jnp.float32),
            pltpu.VMEM((CHUNK, T), jnp.float32),
            pltpu.VMEM((CHUNK, T), jnp.float32),
            pltpu.VMEM((CHUNK,), jnp.int32),
            pltpu.VMEM((CHUNK,), jnp.int32),
            pltpu.VMEM((CHUNK,), jnp.float32),
            pltpu.VMEM((CHUNK,), jnp.float32),
            pltpu.VMEM((CHUNK,), jnp.float32),
            pltpu.VMEM((CHUNK,), jnp.float32),
            pltpu.VMEM((K,), jnp.int32),
        ] + [pltpu.SemaphoreType.DMA] * 10,
    )
    return f(snippets, times, labels, offsets)
